# baseline (speedup 1.0000x reference)
import functools

import jax
import jax.numpy as jnp
from jax import lax
from jax.experimental import pallas as pl
from jax.experimental.pallas import tpu as pltpu
from jax.experimental.pallas import tpu_sc as plsc

N = 10000
D = 128

NC = 2
NS = 16
NW = NC * NS
NP = 10240
RPT = NP // NS
CHUNK = 128

_MESH = plsc.VectorSubcoreMesh(core_axis_name="c", subcore_axis_name="s")


def _copy_idx_row(src2d, row, flat):
    for j in range(CHUNK // 16):
        flat[pl.ds(j * 16, 16)] = src2d[row, pl.ds(j * 16, 16)]


def _zero_vmem_rows(buf, nrows):
    z = jnp.zeros((16,), jnp.float32)
    ncols = buf.shape[1] // 16

    def zr(i, _):
        def zc(j, _):
            buf[i, pl.ds(j * 16, 16)] = z
            return 0
        return lax.fori_loop(0, ncols, zc, 0)

    lax.fori_loop(0, nrows, zr, 0)


def _make_deg_kernel(nchunk):
    npair = nchunk // 2

    @functools.partial(
        pl.kernel,
        out_type=jax.ShapeDtypeStruct((NC, NP, 16), jnp.float32),
        mesh=_MESH,
        scratch_types=[
            pltpu.VMEM((CHUNK,), jnp.int32),
            pltpu.VMEM((CHUNK,), jnp.int32),
            pltpu.VMEM((CHUNK, 16), jnp.float32),
            pltpu.VMEM_SHARED((NP, 16), jnp.float32),
            pltpu.SemaphoreType.DMA,
            pltpu.SemaphoreType.DMA,
        ],
    )
    def deg_kernel(dst_hbm, out_hbm, dflat0, dflat1, ones_v, deg_sh,
                   semA, semB):
        cid = lax.axis_index("c")
        sid = lax.axis_index("s")
        wid = sid * NC + cid
        base0 = wid * nchunk * CHUNK

        def idx_load(i, buf, sem):
            return pltpu.async_copy(
                dst_hbm.at[pl.ds(base0 + i * CHUNK, CHUNK)], buf, sem)

        idx_load(0, dflat0, semA)

        _zero_vmem_rows(ones_v, CHUNK)

        def zs(j, _):
            pltpu.sync_copy(ones_v, deg_sh.at[pl.ds(sid * RPT + j * CHUNK, CHUNK)])
            return 0
        lax.fori_loop(0, RPT // CHUNK, zs, 0)

        one = jnp.ones((16,), jnp.float32)

        def refill(i, _):
            ones_v[i, :] = one
            return 0
        lax.fori_loop(0, CHUNK, refill, 0)

        plsc.subcore_barrier()

        def body(p, _):
            i0 = 2 * p
            idx_load(i0 + 1, dflat1, semB)
            pltpu.make_async_copy(
                dst_hbm.at[pl.ds(base0 + i0 * CHUNK, CHUNK)],
                dflat0, semA).wait()
            pltpu.sync_copy(ones_v, deg_sh.at[dflat0], add=True)

            @pl.when(p < npair - 1)
            def _nxt():
                idx_load(i0 + 2, dflat0, semA)

            pltpu.make_async_copy(
                dst_hbm.at[pl.ds(base0 + (i0 + 1) * CHUNK, CHUNK)],
                dflat1, semB).wait()
            pltpu.sync_copy(ones_v, deg_sh.at[dflat1], add=True)
            return 0
        lax.fori_loop(0, npair, body, 0)

        plsc.subcore_barrier()
        pltpu.sync_copy(deg_sh.at[pl.ds(sid * RPT, RPT)],
                        out_hbm.at[cid, pl.ds(sid * RPT, RPT)])

    return deg_kernel


def _make_edge_kernel(nchunk):
    npair = nchunk // 2

    @functools.partial(
        pl.kernel,
        out_type=jax.ShapeDtypeStruct((NC, NP, D), jnp.float32),
        mesh=_MESH,
        scratch_types=[
            pltpu.VMEM((CHUNK,), jnp.int32),
            pltpu.VMEM((CHUNK,), jnp.int32),
            pltpu.VMEM((CHUNK,), jnp.int32),
            pltpu.VMEM((CHUNK,), jnp.int32),
            pltpu.VMEM((CHUNK, D), jnp.float32),
            pltpu.VMEM_SHARED((NP, D), jnp.float32),
            pltpu.SemaphoreType.DMA,
            pltpu.SemaphoreType.DMA,
            pltpu.SemaphoreType.DMA,
        ],
    )
    def edge_kernel(mp_hbm, src_hbm, dst_hbm, out_hbm,
                    sflat0, sflat1, dflat0, dflat1, rows, agg_sh,
                    semA, semB, sem0):
        cid = lax.axis_index("c")
        sid = lax.axis_index("s")
        wid = sid * NC + cid
        base0 = wid * nchunk * CHUNK

        def idx_load(i, sbuf, dbuf, sem):
            pltpu.async_copy(src_hbm.at[pl.ds(base0 + i * CHUNK, CHUNK)],
                             sbuf, sem)
            pltpu.async_copy(dst_hbm.at[pl.ds(base0 + i * CHUNK, CHUNK)],
                             dbuf, sem)

        def idx_wait(i, sbuf, dbuf, sem):
            pltpu.make_async_copy(src_hbm.at[pl.ds(base0 + i * CHUNK, CHUNK)],
                                  sbuf, sem).wait()
            pltpu.make_async_copy(dst_hbm.at[pl.ds(base0 + i * CHUNK, CHUNK)],
                                  dbuf, sem).wait()

        idx_load(0, sflat0, dflat0, semA)

        _zero_vmem_rows(rows, CHUNK)

        def zs(j, _):
            pltpu.sync_copy(rows, agg_sh.at[pl.ds(sid * RPT + j * CHUNK, CHUNK)])
            return 0
        lax.fori_loop(0, RPT // CHUNK, zs, 0)

        plsc.subcore_barrier()

        def body(p, _):
            i0 = 2 * p
            idx_load(i0 + 1, sflat1, dflat1, semB)
            idx_wait(i0, sflat0, dflat0, semA)
            pltpu.async_copy(mp_hbm.at[sflat0], rows, sem0).wait()
            pltpu.sync_copy(rows, agg_sh.at[dflat0], add=True)

            @pl.when(p < npair - 1)
            def _nxt():
                idx_load(i0 + 2, sflat0, dflat0, semA)

            idx_wait(i0 + 1, sflat1, dflat1, semB)
            pltpu.async_copy(mp_hbm.at[sflat1], rows, sem0).wait()
            pltpu.sync_copy(rows, agg_sh.at[dflat1], add=True)
            return 0
        lax.fori_loop(0, npair, body, 0)

        plsc.subcore_barrier()
        pltpu.sync_copy(agg_sh.at[pl.ds(sid * RPT, RPT)],
                        out_hbm.at[cid, pl.ds(sid * RPT, RPT)])

    return edge_kernel


RB = 1000


def _ln(h, g, b):
    mu = jnp.mean(h, axis=-1, keepdims=True)
    var = jnp.mean((h - mu) ** 2, axis=-1, keepdims=True)
    return (h - mu) * lax.rsqrt(var + 1e-5) * g + b


def _pre_body(x_ref, degp_ref, Wp_, bp_, Wg1_, Wm1_, bm1_, g1_, be1_,
              Wm2_, bm2_, g2_, be2_, Wm3_, bm3_,
              mo_ref, m1p_ref, dinv_ref):
    xb = x_ref[...]
    deg = degp_ref[0, :, 0:1] + degp_ref[1, :, 0:1] + 1.0
    dinv = lax.rsqrt(deg)
    t = jnp.dot(xb, Wm1_[...], preferred_element_type=jnp.float32) + bm1_[...]
    t = jax.nn.relu(_ln(t, g1_[...], be1_[...]))
    t = jnp.dot(t, Wm2_[...], preferred_element_type=jnp.float32) + bm2_[...]
    t = jax.nn.relu(_ln(t, g2_[...], be2_[...]))
    mo_ref[...] = jnp.dot(t, Wm3_[...], preferred_element_type=jnp.float32) + bm3_[...]
    h = jnp.dot(xb, Wp_[...], preferred_element_type=jnp.float32) + bp_[...]
    m1 = jnp.dot(h, Wg1_[...], preferred_element_type=jnp.float32)
    m1p_ref[...] = m1 * dinv
    dinv_ref[...] = dinv


def _mid_body(aggp_ref, m1p_ref, dinv_ref, bg1_, Wg2_, h1_ref, m2p_ref):
    dinv = dinv_ref[...]
    agg1 = dinv * (aggp_ref[0] + aggp_ref[1] + m1p_ref[...])
    h1 = jax.nn.relu(agg1 + bg1_[...])
    h1_ref[...] = h1
    m2 = jnp.dot(h1, Wg2_[...], preferred_element_type=jnp.float32)
    m2p_ref[...] = m2 * dinv


def _post_body(aggp_ref, m2p_ref, dinv_ref, bg2_, h1_ref, Wpred_, bpred_,
               mo_ref, out_ref):
    dinv = dinv_ref[...]
    agg2 = dinv * (aggp_ref[0] + aggp_ref[1] + m2p_ref[...])
    h2 = jax.nn.relu(agg2 + bg2_[...])
    jk = jnp.maximum(h1_ref[...], h2)
    out = jnp.dot(jk, Wpred_[...], preferred_element_type=jnp.float32) + bpred_[...]
    out_ref[...] = out * 0.5 + mo_ref[...] * 0.5


def _row_spec(cols):
    return pl.BlockSpec((RB, cols), lambda i: (i, 0))


def _full_spec(shape):
    nd = len(shape)
    return pl.BlockSpec(shape, lambda i: (0,) * nd)


def _part_spec(cols):
    return pl.BlockSpec((NC, RB, cols), lambda i: (0, i, 0))


def kernel(x, adj, Wp, bp, Wg1, bg1, Wg2, bg2, Wm1, bm1, g1, be1,
           Wm2, bm2, g2, be2, Wm3, bm3, Wpred, bpred):
    n = x.shape[0]
    e = adj.shape[1]
    grid = (n // RB,)

    eb = NW * CHUNK * 2
    e_pad = -(-e // eb) * eb
    src = adj[0]
    dst = adj[1]
    if e_pad != e:
        src = jnp.concatenate([src, jnp.zeros((e_pad - e,), jnp.int32)])
        dst = jnp.concatenate([dst, jnp.full((e_pad - e,), n, jnp.int32)])
    nchunk = e_pad // NW // CHUNK

    deg_k = _make_deg_kernel(nchunk)
    edge_k = _make_edge_kernel(nchunk)

    degp = deg_k(dst)

    bp2 = bp.reshape(1, -1)
    bg1_2 = bg1.reshape(1, -1)
    bg2_2 = bg2.reshape(1, -1)
    bm1_2 = bm1.reshape(1, -1)
    bm2_2 = bm2.reshape(1, -1)
    g1_2 = g1.reshape(1, -1)
    g2_2 = g2.reshape(1, -1)
    be1_2 = be1.reshape(1, -1)
    be2_2 = be2.reshape(1, -1)
    bm3_2 = bm3.reshape(1, 1)
    bpred_2 = bpred.reshape(1, 1)

    mo, m1p, dinv = pl.pallas_call(
        _pre_body,
        grid=grid,
        in_specs=[
            _row_spec(D), _part_spec(16),
            _full_spec((D, D)), _full_spec((1, D)), _full_spec((D, D)),
            _full_spec((D, D)), _full_spec((1, D)), _full_spec((1, D)),
            _full_spec((1, D)),
            _full_spec((D, D)), _full_spec((1, D)), _full_spec((1, D)),
            _full_spec((1, D)),
            _full_spec((D, 1)), _full_spec((1, 1)),
        ],
        out_specs=[_row_spec(1), _row_spec(D), _row_spec(1)],
        out_shape=[
            jax.ShapeDtypeStruct((n, 1), jnp.float32),
            jax.ShapeDtypeStruct((n, D), jnp.float32),
            jax.ShapeDtypeStruct((n, 1), jnp.float32),
        ],
    )(x, degp, Wp, bp2, Wg1, Wm1, bm1_2, g1_2, be1_2,
      Wm2, bm2_2, g2_2, be2_2, Wm3, bm3_2)

    p1 = edge_k(m1p, src, dst)

    h1, m2p = pl.pallas_call(
        _mid_body,
        grid=grid,
        in_specs=[
            _part_spec(D), _row_spec(D), _row_spec(1),
            _full_spec((1, D)), _full_spec((D, D)),
        ],
        out_specs=[_row_spec(D), _row_spec(D)],
        out_shape=[
            jax.ShapeDtypeStruct((n, D), jnp.float32),
            jax.ShapeDtypeStruct((n, D), jnp.float32),
        ],
    )(p1, m1p, dinv, bg1_2, Wg2)

    p2 = edge_k(m2p, src, dst)

    out = pl.pallas_call(
        _post_body,
        grid=grid,
        in_specs=[
            _part_spec(D), _row_spec(D), _row_spec(1),
            _full_spec((1, D)), _row_spec(D),
            _full_spec((D, 1)), _full_spec((1, 1)), _row_spec(1),
        ],
        out_specs=[_row_spec(1)],
        out_shape=[jax.ShapeDtypeStruct((n, 1), jnp.float32)],
    )(p2, m2p, dinv, bg2_2, h1, Wpred, bpred_2, mo)[0]

    return out


# speedup vs baseline: 1.1600x; 1.1600x over previous
import functools

import jax
import jax.numpy as jnp
from jax import lax
from jax.experimental import pallas as pl
from jax.experimental.pallas import tpu as pltpu
from jax.experimental.pallas import tpu_sc as plsc

N = 10000
D = 128

NC = 2
NS = 16
NW = NC * NS
NP = 10240
RPT = NP // NS
CHUNK = 96

_MESH = plsc.VectorSubcoreMesh(core_axis_name="c", subcore_axis_name="s")


def _copy_idx_row(src2d, row, flat):
    for j in range(CHUNK // 16):
        flat[pl.ds(j * 16, 16)] = src2d[row, pl.ds(j * 16, 16)]


def _zero_vmem_rows(buf, nrows):
    z = jnp.zeros((16,), jnp.float32)
    ncols = buf.shape[1] // 16

    def zr(i, _):
        def zc(j, _):
            buf[i, pl.ds(j * 16, 16)] = z
            return 0
        return lax.fori_loop(0, ncols, zc, 0)

    lax.fori_loop(0, nrows, zr, 0)


def _make_deg_kernel(nchunk):
    npair = nchunk // 2

    @functools.partial(
        pl.kernel,
        out_type=jax.ShapeDtypeStruct((NC, NP, 16), jnp.float32),
        mesh=_MESH,
        scratch_types=[
            pltpu.VMEM((CHUNK,), jnp.int32),
            pltpu.VMEM((CHUNK,), jnp.int32),
            pltpu.VMEM((CHUNK, 16), jnp.float32),
            pltpu.VMEM_SHARED((NP, 16), jnp.float32),
            pltpu.SemaphoreType.DMA,
            pltpu.SemaphoreType.DMA,
        ],
    )
    def deg_kernel(dst_hbm, out_hbm, dflat0, dflat1, ones_v, deg_sh,
                   semA, semB):
        cid = lax.axis_index("c")
        sid = lax.axis_index("s")
        wid = sid * NC + cid
        base0 = wid * nchunk * CHUNK

        def idx_load(i, buf, sem):
            return pltpu.async_copy(
                dst_hbm.at[pl.ds(base0 + i * CHUNK, CHUNK)], buf, sem)

        idx_load(0, dflat0, semA)

        _zero_vmem_rows(ones_v, CHUNK)

        def zs(j, _):
            pltpu.sync_copy(ones_v, deg_sh.at[pl.ds(sid * RPT + j * CHUNK, CHUNK)])
            return 0
        lax.fori_loop(0, RPT // CHUNK, zs, 0)

        one = jnp.ones((16,), jnp.float32)

        def refill(i, _):
            ones_v[i, :] = one
            return 0
        lax.fori_loop(0, CHUNK, refill, 0)

        plsc.subcore_barrier()

        def body(p, _):
            i0 = 2 * p
            idx_load(i0 + 1, dflat1, semB)
            pltpu.make_async_copy(
                dst_hbm.at[pl.ds(base0 + i0 * CHUNK, CHUNK)],
                dflat0, semA).wait()
            pltpu.sync_copy(ones_v, deg_sh.at[dflat0], add=True)

            @pl.when(p < npair - 1)
            def _nxt():
                idx_load(i0 + 2, dflat0, semA)

            pltpu.make_async_copy(
                dst_hbm.at[pl.ds(base0 + (i0 + 1) * CHUNK, CHUNK)],
                dflat1, semB).wait()
            pltpu.sync_copy(ones_v, deg_sh.at[dflat1], add=True)
            return 0
        lax.fori_loop(0, npair, body, 0)

        plsc.subcore_barrier()
        pltpu.sync_copy(deg_sh.at[pl.ds(sid * RPT, RPT)],
                        out_hbm.at[cid, pl.ds(sid * RPT, RPT)])

    return deg_kernel


def _make_edge_kernel(nchunk):
    npair = nchunk // 2

    @functools.partial(
        pl.kernel,
        out_type=jax.ShapeDtypeStruct((NC, NP, D), jnp.float32),
        mesh=_MESH,
        scratch_types=[
            pltpu.VMEM((CHUNK,), jnp.int32),
            pltpu.VMEM((CHUNK,), jnp.int32),
            pltpu.VMEM((CHUNK,), jnp.int32),
            pltpu.VMEM((CHUNK,), jnp.int32),
            pltpu.VMEM((CHUNK, D), jnp.float32),
            pltpu.VMEM_SHARED((NP, D), jnp.float32),
            pltpu.SemaphoreType.DMA,
            pltpu.SemaphoreType.DMA,
            pltpu.SemaphoreType.DMA,
        ],
    )
    def edge_kernel(mp_hbm, src_hbm, dst_hbm, out_hbm,
                    sflat0, sflat1, dflat0, dflat1, rows, agg_sh,
                    semA, semB, sem0):
        cid = lax.axis_index("c")
        sid = lax.axis_index("s")
        wid = sid * NC + cid
        base0 = wid * nchunk * CHUNK

        def idx_load(i, sbuf, dbuf, sem):
            pltpu.async_copy(src_hbm.at[pl.ds(base0 + i * CHUNK, CHUNK)],
                             sbuf, sem)
            pltpu.async_copy(dst_hbm.at[pl.ds(base0 + i * CHUNK, CHUNK)],
                             dbuf, sem)

        def idx_wait(i, sbuf, dbuf, sem):
            pltpu.make_async_copy(src_hbm.at[pl.ds(base0 + i * CHUNK, CHUNK)],
                                  sbuf, sem).wait()
            pltpu.make_async_copy(dst_hbm.at[pl.ds(base0 + i * CHUNK, CHUNK)],
                                  dbuf, sem).wait()

        idx_load(0, sflat0, dflat0, semA)

        _zero_vmem_rows(rows, CHUNK)

        def zs(j, _):
            pltpu.sync_copy(rows, agg_sh.at[pl.ds(sid * RPT + j * CHUNK, CHUNK)])
            return 0
        lax.fori_loop(0, RPT // CHUNK, zs, 0)

        plsc.subcore_barrier()

        def body(p, _):
            i0 = 2 * p
            idx_load(i0 + 1, sflat1, dflat1, semB)
            idx_wait(i0, sflat0, dflat0, semA)
            pltpu.async_copy(mp_hbm.at[sflat0], rows, sem0).wait()
            pltpu.sync_copy(rows, agg_sh.at[dflat0], add=True)

            @pl.when(p < npair - 1)
            def _nxt():
                idx_load(i0 + 2, sflat0, dflat0, semA)

            idx_wait(i0 + 1, sflat1, dflat1, semB)
            pltpu.async_copy(mp_hbm.at[sflat1], rows, sem0).wait()
            pltpu.sync_copy(rows, agg_sh.at[dflat1], add=True)
            return 0
        lax.fori_loop(0, npair, body, 0)

        plsc.subcore_barrier()
        pltpu.sync_copy(agg_sh.at[pl.ds(sid * RPT, RPT)],
                        out_hbm.at[cid, pl.ds(sid * RPT, RPT)])

    return edge_kernel


RB = 1000


def _ln(h, g, b):
    mu = jnp.mean(h, axis=-1, keepdims=True)
    var = jnp.mean((h - mu) ** 2, axis=-1, keepdims=True)
    return (h - mu) * lax.rsqrt(var + 1e-5) * g + b


def _pre_body(x_ref, degp_ref, Wp_, bp_, Wg1_, Wm1_, bm1_, g1_, be1_,
              Wm2_, bm2_, g2_, be2_, Wm3_, bm3_,
              mo_ref, m1p_ref, dinv_ref):
    xb = x_ref[...]
    deg = degp_ref[0, :, 0:1] + degp_ref[1, :, 0:1] + 1.0
    dinv = lax.rsqrt(deg)
    t = jnp.dot(xb, Wm1_[...], preferred_element_type=jnp.float32) + bm1_[...]
    t = jax.nn.relu(_ln(t, g1_[...], be1_[...]))
    t = jnp.dot(t, Wm2_[...], preferred_element_type=jnp.float32) + bm2_[...]
    t = jax.nn.relu(_ln(t, g2_[...], be2_[...]))
    mo_ref[...] = jnp.dot(t, Wm3_[...], preferred_element_type=jnp.float32) + bm3_[...]
    h = jnp.dot(xb, Wp_[...], preferred_element_type=jnp.float32) + bp_[...]
    m1 = jnp.dot(h, Wg1_[...], preferred_element_type=jnp.float32)
    m1p_ref[...] = m1 * dinv
    dinv_ref[...] = dinv


def _mid_body(aggp_ref, m1p_ref, dinv_ref, bg1_, Wg2_, h1_ref, m2p_ref):
    dinv = dinv_ref[...]
    agg1 = dinv * (aggp_ref[0] + aggp_ref[1] + m1p_ref[...])
    h1 = jax.nn.relu(agg1 + bg1_[...])
    h1_ref[...] = h1
    m2 = jnp.dot(h1, Wg2_[...], preferred_element_type=jnp.float32)
    m2p_ref[...] = m2 * dinv


def _post_body(aggp_ref, m2p_ref, dinv_ref, bg2_, h1_ref, Wpred_, bpred_,
               mo_ref, out_ref):
    dinv = dinv_ref[...]
    agg2 = dinv * (aggp_ref[0] + aggp_ref[1] + m2p_ref[...])
    h2 = jax.nn.relu(agg2 + bg2_[...])
    jk = jnp.maximum(h1_ref[...], h2)
    out = jnp.dot(jk, Wpred_[...], preferred_element_type=jnp.float32) + bpred_[...]
    out_ref[...] = out * 0.5 + mo_ref[...] * 0.5


def _row_spec(cols):
    return pl.BlockSpec((RB, cols), lambda i: (i, 0))


def _full_spec(shape):
    nd = len(shape)
    return pl.BlockSpec(shape, lambda i: (0,) * nd)


def _part_spec(cols):
    return pl.BlockSpec((NC, RB, cols), lambda i: (0, i, 0))


def kernel(x, adj, Wp, bp, Wg1, bg1, Wg2, bg2, Wm1, bm1, g1, be1,
           Wm2, bm2, g2, be2, Wm3, bm3, Wpred, bpred):
    n = x.shape[0]
    e = adj.shape[1]
    grid = (n // RB,)

    eb = NW * CHUNK * 2
    e_pad = -(-e // eb) * eb
    src = adj[0]
    dst = adj[1]
    if e_pad != e:
        src = jnp.concatenate([src, jnp.zeros((e_pad - e,), jnp.int32)])
        dst = jnp.concatenate([dst, jnp.full((e_pad - e,), n, jnp.int32)])
    nchunk = e_pad // NW // CHUNK

    deg_k = _make_deg_kernel(nchunk)
    edge_k = _make_edge_kernel(nchunk)

    degp = deg_k(dst)

    bp2 = bp.reshape(1, -1)
    bg1_2 = bg1.reshape(1, -1)
    bg2_2 = bg2.reshape(1, -1)
    bm1_2 = bm1.reshape(1, -1)
    bm2_2 = bm2.reshape(1, -1)
    g1_2 = g1.reshape(1, -1)
    g2_2 = g2.reshape(1, -1)
    be1_2 = be1.reshape(1, -1)
    be2_2 = be2.reshape(1, -1)
    bm3_2 = bm3.reshape(1, 1)
    bpred_2 = bpred.reshape(1, 1)

    mo, m1p, dinv = pl.pallas_call(
        _pre_body,
        grid=grid,
        in_specs=[
            _row_spec(D), _part_spec(16),
            _full_spec((D, D)), _full_spec((1, D)), _full_spec((D, D)),
            _full_spec((D, D)), _full_spec((1, D)), _full_spec((1, D)),
            _full_spec((1, D)),
            _full_spec((D, D)), _full_spec((1, D)), _full_spec((1, D)),
            _full_spec((1, D)),
            _full_spec((D, 1)), _full_spec((1, 1)),
        ],
        out_specs=[_row_spec(1), _row_spec(D), _row_spec(1)],
        out_shape=[
            jax.ShapeDtypeStruct((n, 1), jnp.float32),
            jax.ShapeDtypeStruct((n, D), jnp.float32),
            jax.ShapeDtypeStruct((n, 1), jnp.float32),
        ],
    )(x, degp, Wp, bp2, Wg1, Wm1, bm1_2, g1_2, be1_2,
      Wm2, bm2_2, g2_2, be2_2, Wm3, bm3_2)

    p1 = edge_k(m1p, src, dst)

    h1, m2p = pl.pallas_call(
        _mid_body,
        grid=grid,
        in_specs=[
            _part_spec(D), _row_spec(D), _row_spec(1),
            _full_spec((1, D)), _full_spec((D, D)),
        ],
        out_specs=[_row_spec(D), _row_spec(D)],
        out_shape=[
            jax.ShapeDtypeStruct((n, D), jnp.float32),
            jax.ShapeDtypeStruct((n, D), jnp.float32),
        ],
    )(p1, m1p, dinv, bg1_2, Wg2)

    p2 = edge_k(m2p, src, dst)

    out = pl.pallas_call(
        _post_body,
        grid=grid,
        in_specs=[
            _part_spec(D), _row_spec(D), _row_spec(1),
            _full_spec((1, D)), _row_spec(D),
            _full_spec((D, 1)), _full_spec((1, 1)), _row_spec(1),
        ],
        out_specs=[_row_spec(1)],
        out_shape=[jax.ShapeDtypeStruct((n, 1), jnp.float32)],
    )(p2, m2p, dinv, bg2_2, h1, Wpred, bpred_2, mo)[0]

    return out


# speedup vs baseline: 1.4073x; 1.2132x over previous
import functools

import jax
import jax.numpy as jnp
from jax import lax
from jax.experimental import pallas as pl
from jax.experimental.pallas import tpu as pltpu
from jax.experimental.pallas import tpu_sc as plsc

N = 10000
D = 128

NC = 2
NS = 16
NW = NC * NS
NP = 10240
RPT = NP // NS
CHUNK = 64

_MESH = plsc.VectorSubcoreMesh(core_axis_name="c", subcore_axis_name="s")


def _copy_idx_row(src2d, row, flat):
    for j in range(CHUNK // 16):
        flat[pl.ds(j * 16, 16)] = src2d[row, pl.ds(j * 16, 16)]


def _zero_vmem_rows(buf, nrows):
    z = jnp.zeros((16,), jnp.float32)
    ncols = buf.shape[1] // 16

    def zr(i, _):
        def zc(j, _):
            buf[i, pl.ds(j * 16, 16)] = z
            return 0
        return lax.fori_loop(0, ncols, zc, 0)

    lax.fori_loop(0, nrows, zr, 0)


def _make_deg_kernel(nchunk):
    npair = nchunk // 2

    @functools.partial(
        pl.kernel,
        out_type=jax.ShapeDtypeStruct((NC, NP, 16), jnp.float32),
        mesh=_MESH,
        scratch_types=[
            pltpu.VMEM((CHUNK,), jnp.int32),
            pltpu.VMEM((CHUNK,), jnp.int32),
            pltpu.VMEM((CHUNK, 16), jnp.float32),
            pltpu.VMEM_SHARED((NP, 16), jnp.float32),
            pltpu.SemaphoreType.DMA,
            pltpu.SemaphoreType.DMA,
        ],
    )
    def deg_kernel(dst_hbm, out_hbm, dflat0, dflat1, ones_v, deg_sh,
                   semA, semB):
        cid = lax.axis_index("c")
        sid = lax.axis_index("s")
        wid = sid * NC + cid
        base0 = wid * nchunk * CHUNK

        def idx_load(i, buf, sem):
            return pltpu.async_copy(
                dst_hbm.at[pl.ds(base0 + i * CHUNK, CHUNK)], buf, sem)

        idx_load(0, dflat0, semA)

        _zero_vmem_rows(ones_v, CHUNK)

        def zs(j, _):
            pltpu.sync_copy(ones_v, deg_sh.at[pl.ds(sid * RPT + j * CHUNK, CHUNK)])
            return 0
        lax.fori_loop(0, RPT // CHUNK, zs, 0)

        one = jnp.ones((16,), jnp.float32)

        def refill(i, _):
            ones_v[i, :] = one
            return 0
        lax.fori_loop(0, CHUNK, refill, 0)

        plsc.subcore_barrier()

        def body(p, _):
            i0 = 2 * p
            idx_load(i0 + 1, dflat1, semB)
            pltpu.make_async_copy(
                dst_hbm.at[pl.ds(base0 + i0 * CHUNK, CHUNK)],
                dflat0, semA).wait()
            pltpu.sync_copy(ones_v, deg_sh.at[dflat0], add=True)

            @pl.when(p < npair - 1)
            def _nxt():
                idx_load(i0 + 2, dflat0, semA)

            pltpu.make_async_copy(
                dst_hbm.at[pl.ds(base0 + (i0 + 1) * CHUNK, CHUNK)],
                dflat1, semB).wait()
            pltpu.sync_copy(ones_v, deg_sh.at[dflat1], add=True)
            return 0
        lax.fori_loop(0, npair, body, 0)

        plsc.subcore_barrier()
        pltpu.sync_copy(deg_sh.at[pl.ds(sid * RPT, RPT)],
                        out_hbm.at[cid, pl.ds(sid * RPT, RPT)])

    return deg_kernel


def _make_edge_kernel(nchunk):
    npair = nchunk // 2

    @functools.partial(
        pl.kernel,
        out_type=jax.ShapeDtypeStruct((NC, NP, D), jnp.float32),
        mesh=_MESH,
        scratch_types=[
            pltpu.VMEM((CHUNK,), jnp.int32),
            pltpu.VMEM((CHUNK,), jnp.int32),
            pltpu.VMEM((CHUNK,), jnp.int32),
            pltpu.VMEM((CHUNK,), jnp.int32),
            pltpu.VMEM((CHUNK, D), jnp.float32),
            pltpu.VMEM_SHARED((NP, D), jnp.float32),
            pltpu.SemaphoreType.DMA,
            pltpu.SemaphoreType.DMA,
            pltpu.SemaphoreType.DMA,
        ],
    )
    def edge_kernel(mp_hbm, src_hbm, dst_hbm, out_hbm,
                    sflat0, sflat1, dflat0, dflat1, rows, agg_sh,
                    semA, semB, sem0):
        cid = lax.axis_index("c")
        sid = lax.axis_index("s")
        wid = sid * NC + cid
        base0 = wid * nchunk * CHUNK

        def idx_load(i, sbuf, dbuf, sem):
            pltpu.async_copy(src_hbm.at[pl.ds(base0 + i * CHUNK, CHUNK)],
                             sbuf, sem)
            pltpu.async_copy(dst_hbm.at[pl.ds(base0 + i * CHUNK, CHUNK)],
                             dbuf, sem)

        def idx_wait(i, sbuf, dbuf, sem):
            pltpu.make_async_copy(src_hbm.at[pl.ds(base0 + i * CHUNK, CHUNK)],
                                  sbuf, sem).wait()
            pltpu.make_async_copy(dst_hbm.at[pl.ds(base0 + i * CHUNK, CHUNK)],
                                  dbuf, sem).wait()

        idx_load(0, sflat0, dflat0, semA)

        _zero_vmem_rows(rows, CHUNK)

        def zs(j, _):
            pltpu.sync_copy(rows, agg_sh.at[pl.ds(sid * RPT + j * CHUNK, CHUNK)])
            return 0
        lax.fori_loop(0, RPT // CHUNK, zs, 0)

        plsc.subcore_barrier()

        def body(p, _):
            i0 = 2 * p
            idx_load(i0 + 1, sflat1, dflat1, semB)
            idx_wait(i0, sflat0, dflat0, semA)
            pltpu.async_copy(mp_hbm.at[sflat0], rows, sem0).wait()
            pltpu.sync_copy(rows, agg_sh.at[dflat0], add=True)

            @pl.when(p < npair - 1)
            def _nxt():
                idx_load(i0 + 2, sflat0, dflat0, semA)

            idx_wait(i0 + 1, sflat1, dflat1, semB)
            pltpu.async_copy(mp_hbm.at[sflat1], rows, sem0).wait()
            pltpu.sync_copy(rows, agg_sh.at[dflat1], add=True)
            return 0
        lax.fori_loop(0, npair, body, 0)

        plsc.subcore_barrier()
        pltpu.sync_copy(agg_sh.at[pl.ds(sid * RPT, RPT)],
                        out_hbm.at[cid, pl.ds(sid * RPT, RPT)])

    return edge_kernel


RB = 1000


def _ln(h, g, b):
    mu = jnp.mean(h, axis=-1, keepdims=True)
    var = jnp.mean((h - mu) ** 2, axis=-1, keepdims=True)
    return (h - mu) * lax.rsqrt(var + 1e-5) * g + b


def _pre_body(x_ref, degp_ref, Wp_, bp_, Wg1_, Wm1_, bm1_, g1_, be1_,
              Wm2_, bm2_, g2_, be2_, Wm3_, bm3_,
              mo_ref, m1p_ref, dinv_ref):
    xb = x_ref[...]
    deg = degp_ref[0, :, 0:1] + degp_ref[1, :, 0:1] + 1.0
    dinv = lax.rsqrt(deg)
    t = jnp.dot(xb, Wm1_[...], preferred_element_type=jnp.float32) + bm1_[...]
    t = jax.nn.relu(_ln(t, g1_[...], be1_[...]))
    t = jnp.dot(t, Wm2_[...], preferred_element_type=jnp.float32) + bm2_[...]
    t = jax.nn.relu(_ln(t, g2_[...], be2_[...]))
    mo_ref[...] = jnp.dot(t, Wm3_[...], preferred_element_type=jnp.float32) + bm3_[...]
    h = jnp.dot(xb, Wp_[...], preferred_element_type=jnp.float32) + bp_[...]
    m1 = jnp.dot(h, Wg1_[...], preferred_element_type=jnp.float32)
    m1p_ref[...] = m1 * dinv
    dinv_ref[...] = dinv


def _mid_body(aggp_ref, m1p_ref, dinv_ref, bg1_, Wg2_, h1_ref, m2p_ref):
    dinv = dinv_ref[...]
    agg1 = dinv * (aggp_ref[0] + aggp_ref[1] + m1p_ref[...])
    h1 = jax.nn.relu(agg1 + bg1_[...])
    h1_ref[...] = h1
    m2 = jnp.dot(h1, Wg2_[...], preferred_element_type=jnp.float32)
    m2p_ref[...] = m2 * dinv


def _post_body(aggp_ref, m2p_ref, dinv_ref, bg2_, h1_ref, Wpred_, bpred_,
               mo_ref, out_ref):
    dinv = dinv_ref[...]
    agg2 = dinv * (aggp_ref[0] + aggp_ref[1] + m2p_ref[...])
    h2 = jax.nn.relu(agg2 + bg2_[...])
    jk = jnp.maximum(h1_ref[...], h2)
    out = jnp.dot(jk, Wpred_[...], preferred_element_type=jnp.float32) + bpred_[...]
    out_ref[...] = out * 0.5 + mo_ref[...] * 0.5


def _row_spec(cols):
    return pl.BlockSpec((RB, cols), lambda i: (i, 0))


def _full_spec(shape):
    nd = len(shape)
    return pl.BlockSpec(shape, lambda i: (0,) * nd)


def _part_spec(cols):
    return pl.BlockSpec((NC, RB, cols), lambda i: (0, i, 0))


def kernel(x, adj, Wp, bp, Wg1, bg1, Wg2, bg2, Wm1, bm1, g1, be1,
           Wm2, bm2, g2, be2, Wm3, bm3, Wpred, bpred):
    n = x.shape[0]
    e = adj.shape[1]
    grid = (n // RB,)

    eb = NW * CHUNK * 2
    e_pad = -(-e // eb) * eb
    src = adj[0]
    dst = adj[1]
    if e_pad != e:
        src = jnp.concatenate([src, jnp.zeros((e_pad - e,), jnp.int32)])
        dst = jnp.concatenate([dst, jnp.full((e_pad - e,), n, jnp.int32)])
    nchunk = e_pad // NW // CHUNK

    deg_k = _make_deg_kernel(nchunk)
    edge_k = _make_edge_kernel(nchunk)

    degp = deg_k(dst)

    bp2 = bp.reshape(1, -1)
    bg1_2 = bg1.reshape(1, -1)
    bg2_2 = bg2.reshape(1, -1)
    bm1_2 = bm1.reshape(1, -1)
    bm2_2 = bm2.reshape(1, -1)
    g1_2 = g1.reshape(1, -1)
    g2_2 = g2.reshape(1, -1)
    be1_2 = be1.reshape(1, -1)
    be2_2 = be2.reshape(1, -1)
    bm3_2 = bm3.reshape(1, 1)
    bpred_2 = bpred.reshape(1, 1)

    mo, m1p, dinv = pl.pallas_call(
        _pre_body,
        grid=grid,
        in_specs=[
            _row_spec(D), _part_spec(16),
            _full_spec((D, D)), _full_spec((1, D)), _full_spec((D, D)),
            _full_spec((D, D)), _full_spec((1, D)), _full_spec((1, D)),
            _full_spec((1, D)),
            _full_spec((D, D)), _full_spec((1, D)), _full_spec((1, D)),
            _full_spec((1, D)),
            _full_spec((D, 1)), _full_spec((1, 1)),
        ],
        out_specs=[_row_spec(1), _row_spec(D), _row_spec(1)],
        out_shape=[
            jax.ShapeDtypeStruct((n, 1), jnp.float32),
            jax.ShapeDtypeStruct((n, D), jnp.float32),
            jax.ShapeDtypeStruct((n, 1), jnp.float32),
        ],
    )(x, degp, Wp, bp2, Wg1, Wm1, bm1_2, g1_2, be1_2,
      Wm2, bm2_2, g2_2, be2_2, Wm3, bm3_2)

    p1 = edge_k(m1p, src, dst)

    h1, m2p = pl.pallas_call(
        _mid_body,
        grid=grid,
        in_specs=[
            _part_spec(D), _row_spec(D), _row_spec(1),
            _full_spec((1, D)), _full_spec((D, D)),
        ],
        out_specs=[_row_spec(D), _row_spec(D)],
        out_shape=[
            jax.ShapeDtypeStruct((n, D), jnp.float32),
            jax.ShapeDtypeStruct((n, D), jnp.float32),
        ],
    )(p1, m1p, dinv, bg1_2, Wg2)

    p2 = edge_k(m2p, src, dst)

    out = pl.pallas_call(
        _post_body,
        grid=grid,
        in_specs=[
            _part_spec(D), _row_spec(D), _row_spec(1),
            _full_spec((1, D)), _row_spec(D),
            _full_spec((D, 1)), _full_spec((1, 1)), _row_spec(1),
        ],
        out_specs=[_row_spec(1)],
        out_shape=[jax.ShapeDtypeStruct((n, 1), jnp.float32)],
    )(p2, m2p, dinv, bg2_2, h1, Wpred, bpred_2, mo)[0]

    return out


# speedup vs baseline: 1.6493x; 1.1719x over previous
import functools

import jax
import jax.numpy as jnp
from jax import lax
from jax.experimental import pallas as pl
from jax.experimental.pallas import tpu as pltpu
from jax.experimental.pallas import tpu_sc as plsc

N = 10000
D = 128

NC = 2
NS = 16
NW = NC * NS
NP = 10240
RPT = NP // NS
CHUNK = 80

_MESH = plsc.VectorSubcoreMesh(core_axis_name="c", subcore_axis_name="s")


def _copy_idx_row(src2d, row, flat):
    for j in range(CHUNK // 16):
        flat[pl.ds(j * 16, 16)] = src2d[row, pl.ds(j * 16, 16)]


def _zero_vmem_rows(buf, nrows):
    z = jnp.zeros((16,), jnp.float32)
    ncols = buf.shape[1] // 16

    def zr(i, _):
        def zc(j, _):
            buf[i, pl.ds(j * 16, 16)] = z
            return 0
        return lax.fori_loop(0, ncols, zc, 0)

    lax.fori_loop(0, nrows, zr, 0)


def _make_deg_kernel(nchunk):
    npair = nchunk // 2

    @functools.partial(
        pl.kernel,
        out_type=jax.ShapeDtypeStruct((NC, NP, 16), jnp.float32),
        mesh=_MESH,
        scratch_types=[
            pltpu.VMEM((CHUNK,), jnp.int32),
            pltpu.VMEM((CHUNK,), jnp.int32),
            pltpu.VMEM((CHUNK, 16), jnp.float32),
            pltpu.VMEM_SHARED((NP, 16), jnp.float32),
            pltpu.SemaphoreType.DMA,
            pltpu.SemaphoreType.DMA,
        ],
    )
    def deg_kernel(dst_hbm, out_hbm, dflat0, dflat1, ones_v, deg_sh,
                   semA, semB):
        cid = lax.axis_index("c")
        sid = lax.axis_index("s")
        wid = sid * NC + cid
        base0 = wid * nchunk * CHUNK

        def idx_load(i, buf, sem):
            return pltpu.async_copy(
                dst_hbm.at[pl.ds(base0 + i * CHUNK, CHUNK)], buf, sem)

        idx_load(0, dflat0, semA)

        _zero_vmem_rows(ones_v, CHUNK)

        def zs(j, _):
            pltpu.sync_copy(ones_v, deg_sh.at[pl.ds(sid * RPT + j * CHUNK, CHUNK)])
            return 0
        lax.fori_loop(0, RPT // CHUNK, zs, 0)

        one = jnp.ones((16,), jnp.float32)

        def refill(i, _):
            ones_v[i, :] = one
            return 0
        lax.fori_loop(0, CHUNK, refill, 0)

        plsc.subcore_barrier()

        def body(p, _):
            i0 = 2 * p
            idx_load(i0 + 1, dflat1, semB)
            pltpu.make_async_copy(
                dst_hbm.at[pl.ds(base0 + i0 * CHUNK, CHUNK)],
                dflat0, semA).wait()
            pltpu.sync_copy(ones_v, deg_sh.at[dflat0], add=True)

            @pl.when(p < npair - 1)
            def _nxt():
                idx_load(i0 + 2, dflat0, semA)

            pltpu.make_async_copy(
                dst_hbm.at[pl.ds(base0 + (i0 + 1) * CHUNK, CHUNK)],
                dflat1, semB).wait()
            pltpu.sync_copy(ones_v, deg_sh.at[dflat1], add=True)
            return 0
        lax.fori_loop(0, npair, body, 0)

        plsc.subcore_barrier()
        pltpu.sync_copy(deg_sh.at[pl.ds(sid * RPT, RPT)],
                        out_hbm.at[cid, pl.ds(sid * RPT, RPT)])

    return deg_kernel


def _make_edge_kernel(nchunk):
    npair = nchunk // 2

    @functools.partial(
        pl.kernel,
        out_type=jax.ShapeDtypeStruct((NC, NP, D), jnp.float32),
        mesh=_MESH,
        scratch_types=[
            pltpu.VMEM((CHUNK,), jnp.int32),
            pltpu.VMEM((CHUNK,), jnp.int32),
            pltpu.VMEM((CHUNK,), jnp.int32),
            pltpu.VMEM((CHUNK,), jnp.int32),
            pltpu.VMEM((CHUNK, D), jnp.float32),
            pltpu.VMEM_SHARED((NP, D), jnp.float32),
            pltpu.SemaphoreType.DMA,
            pltpu.SemaphoreType.DMA,
            pltpu.SemaphoreType.DMA,
        ],
    )
    def edge_kernel(mp_hbm, src_hbm, dst_hbm, out_hbm,
                    sflat0, sflat1, dflat0, dflat1, rows, agg_sh,
                    semA, semB, sem0):
        cid = lax.axis_index("c")
        sid = lax.axis_index("s")
        wid = sid * NC + cid
        base0 = wid * nchunk * CHUNK

        def idx_load(i, sbuf, dbuf, sem):
            pltpu.async_copy(src_hbm.at[pl.ds(base0 + i * CHUNK, CHUNK)],
                             sbuf, sem)
            pltpu.async_copy(dst_hbm.at[pl.ds(base0 + i * CHUNK, CHUNK)],
                             dbuf, sem)

        def idx_wait(i, sbuf, dbuf, sem):
            pltpu.make_async_copy(src_hbm.at[pl.ds(base0 + i * CHUNK, CHUNK)],
                                  sbuf, sem).wait()
            pltpu.make_async_copy(dst_hbm.at[pl.ds(base0 + i * CHUNK, CHUNK)],
                                  dbuf, sem).wait()

        idx_load(0, sflat0, dflat0, semA)

        _zero_vmem_rows(rows, CHUNK)

        def zs(j, _):
            pltpu.sync_copy(rows, agg_sh.at[pl.ds(sid * RPT + j * CHUNK, CHUNK)])
            return 0
        lax.fori_loop(0, RPT // CHUNK, zs, 0)

        plsc.subcore_barrier()

        def body(p, _):
            i0 = 2 * p
            idx_load(i0 + 1, sflat1, dflat1, semB)
            idx_wait(i0, sflat0, dflat0, semA)
            pltpu.async_copy(mp_hbm.at[sflat0], rows, sem0).wait()
            pltpu.sync_copy(rows, agg_sh.at[dflat0], add=True)

            @pl.when(p < npair - 1)
            def _nxt():
                idx_load(i0 + 2, sflat0, dflat0, semA)

            idx_wait(i0 + 1, sflat1, dflat1, semB)
            pltpu.async_copy(mp_hbm.at[sflat1], rows, sem0).wait()
            pltpu.sync_copy(rows, agg_sh.at[dflat1], add=True)
            return 0
        lax.fori_loop(0, npair, body, 0)

        plsc.subcore_barrier()
        pltpu.sync_copy(agg_sh.at[pl.ds(sid * RPT, RPT)],
                        out_hbm.at[cid, pl.ds(sid * RPT, RPT)])

    return edge_kernel


RB = 1000


def _ln(h, g, b):
    mu = jnp.mean(h, axis=-1, keepdims=True)
    var = jnp.mean((h - mu) ** 2, axis=-1, keepdims=True)
    return (h - mu) * lax.rsqrt(var + 1e-5) * g + b


def _pre_body(x_ref, degp_ref, Wp_, bp_, Wg1_, Wm1_, bm1_, g1_, be1_,
              Wm2_, bm2_, g2_, be2_, Wm3_, bm3_,
              mo_ref, m1p_ref, dinv_ref):
    xb = x_ref[...]
    deg = degp_ref[0, :, 0:1] + degp_ref[1, :, 0:1] + 1.0
    dinv = lax.rsqrt(deg)
    t = jnp.dot(xb, Wm1_[...], preferred_element_type=jnp.float32) + bm1_[...]
    t = jax.nn.relu(_ln(t, g1_[...], be1_[...]))
    t = jnp.dot(t, Wm2_[...], preferred_element_type=jnp.float32) + bm2_[...]
    t = jax.nn.relu(_ln(t, g2_[...], be2_[...]))
    mo_ref[...] = jnp.dot(t, Wm3_[...], preferred_element_type=jnp.float32) + bm3_[...]
    h = jnp.dot(xb, Wp_[...], preferred_element_type=jnp.float32) + bp_[...]
    m1 = jnp.dot(h, Wg1_[...], preferred_element_type=jnp.float32)
    m1p_ref[...] = m1 * dinv
    dinv_ref[...] = dinv


def _mid_body(aggp_ref, m1p_ref, dinv_ref, bg1_, Wg2_, h1_ref, m2p_ref):
    dinv = dinv_ref[...]
    agg1 = dinv * (aggp_ref[0] + aggp_ref[1] + m1p_ref[...])
    h1 = jax.nn.relu(agg1 + bg1_[...])
    h1_ref[...] = h1
    m2 = jnp.dot(h1, Wg2_[...], preferred_element_type=jnp.float32)
    m2p_ref[...] = m2 * dinv


def _post_body(aggp_ref, m2p_ref, dinv_ref, bg2_, h1_ref, Wpred_, bpred_,
               mo_ref, out_ref):
    dinv = dinv_ref[...]
    agg2 = dinv * (aggp_ref[0] + aggp_ref[1] + m2p_ref[...])
    h2 = jax.nn.relu(agg2 + bg2_[...])
    jk = jnp.maximum(h1_ref[...], h2)
    out = jnp.dot(jk, Wpred_[...], preferred_element_type=jnp.float32) + bpred_[...]
    out_ref[...] = out * 0.5 + mo_ref[...] * 0.5


def _row_spec(cols):
    return pl.BlockSpec((RB, cols), lambda i: (i, 0))


def _full_spec(shape):
    nd = len(shape)
    return pl.BlockSpec(shape, lambda i: (0,) * nd)


def _part_spec(cols):
    return pl.BlockSpec((NC, RB, cols), lambda i: (0, i, 0))


def kernel(x, adj, Wp, bp, Wg1, bg1, Wg2, bg2, Wm1, bm1, g1, be1,
           Wm2, bm2, g2, be2, Wm3, bm3, Wpred, bpred):
    n = x.shape[0]
    e = adj.shape[1]
    grid = (n // RB,)

    eb = NW * CHUNK * 2
    e_pad = -(-e // eb) * eb
    src = adj[0]
    dst = adj[1]
    if e_pad != e:
        src = jnp.concatenate([src, jnp.zeros((e_pad - e,), jnp.int32)])
        dst = jnp.concatenate([dst, jnp.full((e_pad - e,), n, jnp.int32)])
    nchunk = e_pad // NW // CHUNK

    deg_k = _make_deg_kernel(nchunk)
    edge_k = _make_edge_kernel(nchunk)

    degp = deg_k(dst)

    bp2 = bp.reshape(1, -1)
    bg1_2 = bg1.reshape(1, -1)
    bg2_2 = bg2.reshape(1, -1)
    bm1_2 = bm1.reshape(1, -1)
    bm2_2 = bm2.reshape(1, -1)
    g1_2 = g1.reshape(1, -1)
    g2_2 = g2.reshape(1, -1)
    be1_2 = be1.reshape(1, -1)
    be2_2 = be2.reshape(1, -1)
    bm3_2 = bm3.reshape(1, 1)
    bpred_2 = bpred.reshape(1, 1)

    mo, m1p, dinv = pl.pallas_call(
        _pre_body,
        grid=grid,
        in_specs=[
            _row_spec(D), _part_spec(16),
            _full_spec((D, D)), _full_spec((1, D)), _full_spec((D, D)),
            _full_spec((D, D)), _full_spec((1, D)), _full_spec((1, D)),
            _full_spec((1, D)),
            _full_spec((D, D)), _full_spec((1, D)), _full_spec((1, D)),
            _full_spec((1, D)),
            _full_spec((D, 1)), _full_spec((1, 1)),
        ],
        out_specs=[_row_spec(1), _row_spec(D), _row_spec(1)],
        out_shape=[
            jax.ShapeDtypeStruct((n, 1), jnp.float32),
            jax.ShapeDtypeStruct((n, D), jnp.float32),
            jax.ShapeDtypeStruct((n, 1), jnp.float32),
        ],
    )(x, degp, Wp, bp2, Wg1, Wm1, bm1_2, g1_2, be1_2,
      Wm2, bm2_2, g2_2, be2_2, Wm3, bm3_2)

    p1 = edge_k(m1p, src, dst)

    h1, m2p = pl.pallas_call(
        _mid_body,
        grid=grid,
        in_specs=[
            _part_spec(D), _row_spec(D), _row_spec(1),
            _full_spec((1, D)), _full_spec((D, D)),
        ],
        out_specs=[_row_spec(D), _row_spec(D)],
        out_shape=[
            jax.ShapeDtypeStruct((n, D), jnp.float32),
            jax.ShapeDtypeStruct((n, D), jnp.float32),
        ],
    )(p1, m1p, dinv, bg1_2, Wg2)

    p2 = edge_k(m2p, src, dst)

    out = pl.pallas_call(
        _post_body,
        grid=grid,
        in_specs=[
            _part_spec(D), _row_spec(D), _row_spec(1),
            _full_spec((1, D)), _row_spec(D),
            _full_spec((D, 1)), _full_spec((1, 1)), _row_spec(1),
        ],
        out_specs=[_row_spec(1)],
        out_shape=[jax.ShapeDtypeStruct((n, 1), jnp.float32)],
    )(p2, m2p, dinv, bg2_2, h1, Wpred, bpred_2, mo)[0]

    return out


# speedup vs baseline: 1.7700x; 1.0732x over previous
import functools

import jax
import jax.numpy as jnp
from jax import lax
from jax.experimental import pallas as pl
from jax.experimental.pallas import tpu as pltpu
from jax.experimental.pallas import tpu_sc as plsc

N = 10000
D = 128

NC = 2
NS = 16
NW = NC * NS
NP = 10240
RPT = NP // NS
CHUNK = 80

_MESH = plsc.VectorSubcoreMesh(core_axis_name="c", subcore_axis_name="s")


def _copy_idx_row(src2d, row, flat):
    for j in range(CHUNK // 16):
        flat[pl.ds(j * 16, 16)] = src2d[row, pl.ds(j * 16, 16)]


def _zero_vmem_rows(buf, nrows):
    z = jnp.zeros((16,), jnp.float32)
    ncols = buf.shape[1] // 16

    def zr(i, _):
        def zc(j, _):
            buf[i, pl.ds(j * 16, 16)] = z
            return 0
        return lax.fori_loop(0, ncols, zc, 0)

    lax.fori_loop(0, nrows, zr, 0)


def _make_deg_kernel(nchunk):
    npair = nchunk // 2

    @functools.partial(
        pl.kernel,
        out_type=jax.ShapeDtypeStruct((NC, NP, 16), jnp.float32),
        mesh=_MESH,
        scratch_types=[
            pltpu.VMEM((CHUNK,), jnp.int32),
            pltpu.VMEM((CHUNK,), jnp.int32),
            pltpu.VMEM((CHUNK, 16), jnp.float32),
            pltpu.VMEM_SHARED((NP, 16), jnp.float32),
            pltpu.SemaphoreType.DMA,
            pltpu.SemaphoreType.DMA,
        ],
    )
    def deg_kernel(dst_hbm, out_hbm, dflat0, dflat1, ones_v, deg_sh,
                   semA, semB):
        cid = lax.axis_index("c")
        sid = lax.axis_index("s")
        wid = sid * NC + cid
        base0 = wid * nchunk * CHUNK

        def idx_load(i, buf, sem):
            return pltpu.async_copy(
                dst_hbm.at[pl.ds(base0 + i * CHUNK, CHUNK)], buf, sem)

        idx_load(0, dflat0, semA)

        _zero_vmem_rows(ones_v, CHUNK)

        def zs(j, _):
            pltpu.sync_copy(ones_v, deg_sh.at[pl.ds(sid * RPT + j * CHUNK, CHUNK)])
            return 0
        lax.fori_loop(0, RPT // CHUNK, zs, 0)

        one = jnp.ones((16,), jnp.float32)

        def refill(i, _):
            ones_v[i, :] = one
            return 0
        lax.fori_loop(0, CHUNK, refill, 0)

        plsc.subcore_barrier()

        def body(p, _):
            i0 = 2 * p
            idx_load(i0 + 1, dflat1, semB)
            pltpu.make_async_copy(
                dst_hbm.at[pl.ds(base0 + i0 * CHUNK, CHUNK)],
                dflat0, semA).wait()
            pltpu.sync_copy(ones_v, deg_sh.at[dflat0], add=True)

            @pl.when(p < npair - 1)
            def _nxt():
                idx_load(i0 + 2, dflat0, semA)

            pltpu.make_async_copy(
                dst_hbm.at[pl.ds(base0 + (i0 + 1) * CHUNK, CHUNK)],
                dflat1, semB).wait()
            pltpu.sync_copy(ones_v, deg_sh.at[dflat1], add=True)
            return 0
        lax.fori_loop(0, npair, body, 0)

        plsc.subcore_barrier()
        pltpu.sync_copy(deg_sh.at[pl.ds(sid * RPT, RPT)],
                        out_hbm.at[cid, pl.ds(sid * RPT, RPT)])

    return deg_kernel


FRAC0 = 0.58


def _make_edge_kernel(nchunk):
    total_pc = nchunk * NC
    k0 = int(total_pc * FRAC0) // 2 * 2
    k1 = total_pc - k0

    @functools.partial(
        pl.kernel,
        out_type=jax.ShapeDtypeStruct((NC, NP, D), jnp.float32),
        mesh=_MESH,
        scratch_types=[
            pltpu.VMEM((CHUNK,), jnp.int32),
            pltpu.VMEM((CHUNK,), jnp.int32),
            pltpu.VMEM((CHUNK,), jnp.int32),
            pltpu.VMEM((CHUNK,), jnp.int32),
            pltpu.VMEM((CHUNK, D), jnp.float32),
            pltpu.VMEM_SHARED((NP, D), jnp.float32),
            pltpu.SemaphoreType.DMA,
            pltpu.SemaphoreType.DMA,
            pltpu.SemaphoreType.DMA,
        ],
    )
    def edge_kernel(mp_hbm, src_hbm, dst_hbm, out_hbm,
                    sflat0, sflat1, dflat0, dflat1, rows, agg_sh,
                    semA, semB, sem0):
        cid = lax.axis_index("c")
        sid = lax.axis_index("s")
        base0 = jnp.where(cid == 0, sid * k0, NS * k0 + sid * k1) * CHUNK
        npair = jnp.where(cid == 0, k0 // 2, k1 // 2)

        def idx_load(i, sbuf, dbuf, sem):
            pltpu.async_copy(src_hbm.at[pl.ds(base0 + i * CHUNK, CHUNK)],
                             sbuf, sem)
            pltpu.async_copy(dst_hbm.at[pl.ds(base0 + i * CHUNK, CHUNK)],
                             dbuf, sem)

        def idx_wait(i, sbuf, dbuf, sem):
            pltpu.make_async_copy(src_hbm.at[pl.ds(base0 + i * CHUNK, CHUNK)],
                                  sbuf, sem).wait()
            pltpu.make_async_copy(dst_hbm.at[pl.ds(base0 + i * CHUNK, CHUNK)],
                                  dbuf, sem).wait()

        idx_load(0, sflat0, dflat0, semA)

        _zero_vmem_rows(rows, CHUNK)

        def zs(j, _):
            pltpu.sync_copy(rows, agg_sh.at[pl.ds(sid * RPT + j * CHUNK, CHUNK)])
            return 0
        lax.fori_loop(0, RPT // CHUNK, zs, 0)

        plsc.subcore_barrier()

        def body(p, _):
            i0 = 2 * p
            idx_load(i0 + 1, sflat1, dflat1, semB)
            idx_wait(i0, sflat0, dflat0, semA)
            pltpu.async_copy(mp_hbm.at[sflat0], rows, sem0).wait()
            pltpu.sync_copy(rows, agg_sh.at[dflat0], add=True)

            @pl.when(p < npair - 1)
            def _nxt():
                idx_load(i0 + 2, sflat0, dflat0, semA)

            idx_wait(i0 + 1, sflat1, dflat1, semB)
            pltpu.async_copy(mp_hbm.at[sflat1], rows, sem0).wait()
            pltpu.sync_copy(rows, agg_sh.at[dflat1], add=True)
            return 0
        lax.fori_loop(0, npair, body, 0)

        plsc.subcore_barrier()
        pltpu.sync_copy(agg_sh.at[pl.ds(sid * RPT, RPT)],
                        out_hbm.at[cid, pl.ds(sid * RPT, RPT)])

    return edge_kernel


RB = 1000


def _ln(h, g, b):
    mu = jnp.mean(h, axis=-1, keepdims=True)
    var = jnp.mean((h - mu) ** 2, axis=-1, keepdims=True)
    return (h - mu) * lax.rsqrt(var + 1e-5) * g + b


def _pre_body(x_ref, degp_ref, Wp_, bp_, Wg1_, Wm1_, bm1_, g1_, be1_,
              Wm2_, bm2_, g2_, be2_, Wm3_, bm3_,
              mo_ref, m1p_ref, dinv_ref):
    xb = x_ref[...]
    deg = degp_ref[0, :, 0:1] + degp_ref[1, :, 0:1] + 1.0
    dinv = lax.rsqrt(deg)
    t = jnp.dot(xb, Wm1_[...], preferred_element_type=jnp.float32) + bm1_[...]
    t = jax.nn.relu(_ln(t, g1_[...], be1_[...]))
    t = jnp.dot(t, Wm2_[...], preferred_element_type=jnp.float32) + bm2_[...]
    t = jax.nn.relu(_ln(t, g2_[...], be2_[...]))
    mo_ref[...] = jnp.dot(t, Wm3_[...], preferred_element_type=jnp.float32) + bm3_[...]
    h = jnp.dot(xb, Wp_[...], preferred_element_type=jnp.float32) + bp_[...]
    m1 = jnp.dot(h, Wg1_[...], preferred_element_type=jnp.float32)
    m1p_ref[...] = m1 * dinv
    dinv_ref[...] = dinv


def _mid_body(aggp_ref, m1p_ref, dinv_ref, bg1_, Wg2_, h1_ref, m2p_ref):
    dinv = dinv_ref[...]
    agg1 = dinv * (aggp_ref[0] + aggp_ref[1] + m1p_ref[...])
    h1 = jax.nn.relu(agg1 + bg1_[...])
    h1_ref[...] = h1
    m2 = jnp.dot(h1, Wg2_[...], preferred_element_type=jnp.float32)
    m2p_ref[...] = m2 * dinv


def _post_body(aggp_ref, m2p_ref, dinv_ref, bg2_, h1_ref, Wpred_, bpred_,
               mo_ref, out_ref):
    dinv = dinv_ref[...]
    agg2 = dinv * (aggp_ref[0] + aggp_ref[1] + m2p_ref[...])
    h2 = jax.nn.relu(agg2 + bg2_[...])
    jk = jnp.maximum(h1_ref[...], h2)
    out = jnp.dot(jk, Wpred_[...], preferred_element_type=jnp.float32) + bpred_[...]
    out_ref[...] = out * 0.5 + mo_ref[...] * 0.5


def _row_spec(cols):
    return pl.BlockSpec((RB, cols), lambda i: (i, 0))


def _full_spec(shape):
    nd = len(shape)
    return pl.BlockSpec(shape, lambda i: (0,) * nd)


def _part_spec(cols):
    return pl.BlockSpec((NC, RB, cols), lambda i: (0, i, 0))


def kernel(x, adj, Wp, bp, Wg1, bg1, Wg2, bg2, Wm1, bm1, g1, be1,
           Wm2, bm2, g2, be2, Wm3, bm3, Wpred, bpred):
    n = x.shape[0]
    e = adj.shape[1]
    grid = (n // RB,)

    eb = NW * CHUNK * 2
    e_pad = -(-e // eb) * eb
    src = adj[0]
    dst = adj[1]
    if e_pad != e:
        src = jnp.concatenate([src, jnp.zeros((e_pad - e,), jnp.int32)])
        dst = jnp.concatenate([dst, jnp.full((e_pad - e,), n, jnp.int32)])
    nchunk = e_pad // NW // CHUNK

    deg_k = _make_deg_kernel(nchunk)
    edge_k = _make_edge_kernel(nchunk)

    degp = deg_k(dst)

    bp2 = bp.reshape(1, -1)
    bg1_2 = bg1.reshape(1, -1)
    bg2_2 = bg2.reshape(1, -1)
    bm1_2 = bm1.reshape(1, -1)
    bm2_2 = bm2.reshape(1, -1)
    g1_2 = g1.reshape(1, -1)
    g2_2 = g2.reshape(1, -1)
    be1_2 = be1.reshape(1, -1)
    be2_2 = be2.reshape(1, -1)
    bm3_2 = bm3.reshape(1, 1)
    bpred_2 = bpred.reshape(1, 1)

    mo, m1p, dinv = pl.pallas_call(
        _pre_body,
        grid=grid,
        in_specs=[
            _row_spec(D), _part_spec(16),
            _full_spec((D, D)), _full_spec((1, D)), _full_spec((D, D)),
            _full_spec((D, D)), _full_spec((1, D)), _full_spec((1, D)),
            _full_spec((1, D)),
            _full_spec((D, D)), _full_spec((1, D)), _full_spec((1, D)),
            _full_spec((1, D)),
            _full_spec((D, 1)), _full_spec((1, 1)),
        ],
        out_specs=[_row_spec(1), _row_spec(D), _row_spec(1)],
        out_shape=[
            jax.ShapeDtypeStruct((n, 1), jnp.float32),
            jax.ShapeDtypeStruct((n, D), jnp.float32),
            jax.ShapeDtypeStruct((n, 1), jnp.float32),
        ],
    )(x, degp, Wp, bp2, Wg1, Wm1, bm1_2, g1_2, be1_2,
      Wm2, bm2_2, g2_2, be2_2, Wm3, bm3_2)

    p1 = edge_k(m1p, src, dst)

    h1, m2p = pl.pallas_call(
        _mid_body,
        grid=grid,
        in_specs=[
            _part_spec(D), _row_spec(D), _row_spec(1),
            _full_spec((1, D)), _full_spec((D, D)),
        ],
        out_specs=[_row_spec(D), _row_spec(D)],
        out_shape=[
            jax.ShapeDtypeStruct((n, D), jnp.float32),
            jax.ShapeDtypeStruct((n, D), jnp.float32),
        ],
    )(p1, m1p, dinv, bg1_2, Wg2)

    p2 = edge_k(m2p, src, dst)

    out = pl.pallas_call(
        _post_body,
        grid=grid,
        in_specs=[
            _part_spec(D), _row_spec(D), _row_spec(1),
            _full_spec((1, D)), _row_spec(D),
            _full_spec((D, 1)), _full_spec((1, 1)), _row_spec(1),
        ],
        out_specs=[_row_spec(1)],
        out_shape=[jax.ShapeDtypeStruct((n, 1), jnp.float32)],
    )(p2, m2p, dinv, bg2_2, h1, Wpred, bpred_2, mo)[0]

    return out


# speedup vs baseline: 1.7734x; 1.0019x over previous
import functools

import jax
import jax.numpy as jnp
from jax import lax
from jax.experimental import pallas as pl
from jax.experimental.pallas import tpu as pltpu
from jax.experimental.pallas import tpu_sc as plsc

N = 10000
D = 128

NC = 2
NS = 16
NW = NC * NS
NP = 10240
RPT = NP // NS
CHUNK = 80

_MESH = plsc.VectorSubcoreMesh(core_axis_name="c", subcore_axis_name="s")


def _copy_idx_row(src2d, row, flat):
    for j in range(CHUNK // 16):
        flat[pl.ds(j * 16, 16)] = src2d[row, pl.ds(j * 16, 16)]


def _zero_vmem_rows(buf, nrows):
    z = jnp.zeros((16,), jnp.float32)
    ncols = buf.shape[1] // 16

    def zr(i, _):
        def zc(j, _):
            buf[i, pl.ds(j * 16, 16)] = z
            return 0
        return lax.fori_loop(0, ncols, zc, 0)

    lax.fori_loop(0, nrows, zr, 0)


def _make_deg_kernel(nchunk):
    npair = nchunk // 2

    @functools.partial(
        pl.kernel,
        out_type=jax.ShapeDtypeStruct((NC, NP, 16), jnp.float32),
        mesh=_MESH,
        scratch_types=[
            pltpu.VMEM((CHUNK,), jnp.int32),
            pltpu.VMEM((CHUNK,), jnp.int32),
            pltpu.VMEM((CHUNK, 16), jnp.float32),
            pltpu.VMEM_SHARED((NP, 16), jnp.float32),
            pltpu.SemaphoreType.DMA,
            pltpu.SemaphoreType.DMA,
        ],
    )
    def deg_kernel(dst_hbm, out_hbm, dflat0, dflat1, ones_v, deg_sh,
                   semA, semB):
        cid = lax.axis_index("c")
        sid = lax.axis_index("s")
        wid = sid * NC + cid
        base0 = wid * nchunk * CHUNK

        def idx_load(i, buf, sem):
            return pltpu.async_copy(
                dst_hbm.at[pl.ds(base0 + i * CHUNK, CHUNK)], buf, sem)

        idx_load(0, dflat0, semA)

        _zero_vmem_rows(ones_v, CHUNK)

        def zs(j, _):
            pltpu.sync_copy(ones_v, deg_sh.at[pl.ds(sid * RPT + j * CHUNK, CHUNK)])
            return 0
        lax.fori_loop(0, RPT // CHUNK, zs, 0)

        one = jnp.ones((16,), jnp.float32)

        def refill(i, _):
            ones_v[i, :] = one
            return 0
        lax.fori_loop(0, CHUNK, refill, 0)

        plsc.subcore_barrier()

        def body(p, _):
            i0 = 2 * p
            idx_load(i0 + 1, dflat1, semB)
            pltpu.make_async_copy(
                dst_hbm.at[pl.ds(base0 + i0 * CHUNK, CHUNK)],
                dflat0, semA).wait()
            pltpu.sync_copy(ones_v, deg_sh.at[dflat0], add=True)

            @pl.when(p < npair - 1)
            def _nxt():
                idx_load(i0 + 2, dflat0, semA)

            pltpu.make_async_copy(
                dst_hbm.at[pl.ds(base0 + (i0 + 1) * CHUNK, CHUNK)],
                dflat1, semB).wait()
            pltpu.sync_copy(ones_v, deg_sh.at[dflat1], add=True)
            return 0
        lax.fori_loop(0, npair, body, 0)

        plsc.subcore_barrier()
        pltpu.sync_copy(deg_sh.at[pl.ds(sid * RPT, RPT)],
                        out_hbm.at[cid, pl.ds(sid * RPT, RPT)])

    return deg_kernel


FRAC0 = 0.62


def _make_edge_kernel(nchunk):
    total_pc = nchunk * NC
    k0 = int(total_pc * FRAC0) // 2 * 2
    k1 = total_pc - k0

    @functools.partial(
        pl.kernel,
        out_type=jax.ShapeDtypeStruct((NC, NP, D), jnp.float32),
        mesh=_MESH,
        scratch_types=[
            pltpu.VMEM((CHUNK,), jnp.int32),
            pltpu.VMEM((CHUNK,), jnp.int32),
            pltpu.VMEM((CHUNK,), jnp.int32),
            pltpu.VMEM((CHUNK,), jnp.int32),
            pltpu.VMEM((CHUNK, D), jnp.float32),
            pltpu.VMEM_SHARED((NP, D), jnp.float32),
            pltpu.SemaphoreType.DMA,
            pltpu.SemaphoreType.DMA,
            pltpu.SemaphoreType.DMA,
        ],
    )
    def edge_kernel(mp_hbm, src_hbm, dst_hbm, out_hbm,
                    sflat0, sflat1, dflat0, dflat1, rows, agg_sh,
                    semA, semB, sem0):
        cid = lax.axis_index("c")
        sid = lax.axis_index("s")
        base0 = jnp.where(cid == 0, sid * k0, NS * k0 + sid * k1) * CHUNK
        npair = jnp.where(cid == 0, k0 // 2, k1 // 2)

        def idx_load(i, sbuf, dbuf, sem):
            pltpu.async_copy(src_hbm.at[pl.ds(base0 + i * CHUNK, CHUNK)],
                             sbuf, sem)
            pltpu.async_copy(dst_hbm.at[pl.ds(base0 + i * CHUNK, CHUNK)],
                             dbuf, sem)

        def idx_wait(i, sbuf, dbuf, sem):
            pltpu.make_async_copy(src_hbm.at[pl.ds(base0 + i * CHUNK, CHUNK)],
                                  sbuf, sem).wait()
            pltpu.make_async_copy(dst_hbm.at[pl.ds(base0 + i * CHUNK, CHUNK)],
                                  dbuf, sem).wait()

        idx_load(0, sflat0, dflat0, semA)

        _zero_vmem_rows(rows, CHUNK)

        def zs(j, _):
            pltpu.sync_copy(rows, agg_sh.at[pl.ds(sid * RPT + j * CHUNK, CHUNK)])
            return 0
        lax.fori_loop(0, RPT // CHUNK, zs, 0)

        plsc.subcore_barrier()

        def body(p, _):
            i0 = 2 * p
            idx_load(i0 + 1, sflat1, dflat1, semB)
            idx_wait(i0, sflat0, dflat0, semA)
            pltpu.async_copy(mp_hbm.at[sflat0], rows, sem0).wait()
            pltpu.sync_copy(rows, agg_sh.at[dflat0], add=True)

            @pl.when(p < npair - 1)
            def _nxt():
                idx_load(i0 + 2, sflat0, dflat0, semA)

            idx_wait(i0 + 1, sflat1, dflat1, semB)
            pltpu.async_copy(mp_hbm.at[sflat1], rows, sem0).wait()
            pltpu.sync_copy(rows, agg_sh.at[dflat1], add=True)
            return 0
        lax.fori_loop(0, npair, body, 0)

        plsc.subcore_barrier()
        pltpu.sync_copy(agg_sh.at[pl.ds(sid * RPT, RPT)],
                        out_hbm.at[cid, pl.ds(sid * RPT, RPT)])

    return edge_kernel


RB = 1000


def _ln(h, g, b):
    mu = jnp.mean(h, axis=-1, keepdims=True)
    var = jnp.mean((h - mu) ** 2, axis=-1, keepdims=True)
    return (h - mu) * lax.rsqrt(var + 1e-5) * g + b


def _pre_body(x_ref, degp_ref, Wp_, bp_, Wg1_, Wm1_, bm1_, g1_, be1_,
              Wm2_, bm2_, g2_, be2_, Wm3_, bm3_,
              mo_ref, m1p_ref, dinv_ref):
    xb = x_ref[...]
    deg = degp_ref[0, :, 0:1] + degp_ref[1, :, 0:1] + 1.0
    dinv = lax.rsqrt(deg)
    t = jnp.dot(xb, Wm1_[...], preferred_element_type=jnp.float32) + bm1_[...]
    t = jax.nn.relu(_ln(t, g1_[...], be1_[...]))
    t = jnp.dot(t, Wm2_[...], preferred_element_type=jnp.float32) + bm2_[...]
    t = jax.nn.relu(_ln(t, g2_[...], be2_[...]))
    mo_ref[...] = jnp.dot(t, Wm3_[...], preferred_element_type=jnp.float32) + bm3_[...]
    h = jnp.dot(xb, Wp_[...], preferred_element_type=jnp.float32) + bp_[...]
    m1 = jnp.dot(h, Wg1_[...], preferred_element_type=jnp.float32)
    m1p_ref[...] = m1 * dinv
    dinv_ref[...] = dinv


def _mid_body(aggp_ref, m1p_ref, dinv_ref, bg1_, Wg2_, h1_ref, m2p_ref):
    dinv = dinv_ref[...]
    agg1 = dinv * (aggp_ref[0] + aggp_ref[1] + m1p_ref[...])
    h1 = jax.nn.relu(agg1 + bg1_[...])
    h1_ref[...] = h1
    m2 = jnp.dot(h1, Wg2_[...], preferred_element_type=jnp.float32)
    m2p_ref[...] = m2 * dinv


def _post_body(aggp_ref, m2p_ref, dinv_ref, bg2_, h1_ref, Wpred_, bpred_,
               mo_ref, out_ref):
    dinv = dinv_ref[...]
    agg2 = dinv * (aggp_ref[0] + aggp_ref[1] + m2p_ref[...])
    h2 = jax.nn.relu(agg2 + bg2_[...])
    jk = jnp.maximum(h1_ref[...], h2)
    out = jnp.dot(jk, Wpred_[...], preferred_element_type=jnp.float32) + bpred_[...]
    out_ref[...] = out * 0.5 + mo_ref[...] * 0.5


def _row_spec(cols):
    return pl.BlockSpec((RB, cols), lambda i: (i, 0))


def _full_spec(shape):
    nd = len(shape)
    return pl.BlockSpec(shape, lambda i: (0,) * nd)


def _part_spec(cols):
    return pl.BlockSpec((NC, RB, cols), lambda i: (0, i, 0))


def kernel(x, adj, Wp, bp, Wg1, bg1, Wg2, bg2, Wm1, bm1, g1, be1,
           Wm2, bm2, g2, be2, Wm3, bm3, Wpred, bpred):
    n = x.shape[0]
    e = adj.shape[1]
    grid = (n // RB,)

    eb = NW * CHUNK * 2
    e_pad = -(-e // eb) * eb
    src = adj[0]
    dst = adj[1]
    if e_pad != e:
        src = jnp.concatenate([src, jnp.zeros((e_pad - e,), jnp.int32)])
        dst = jnp.concatenate([dst, jnp.full((e_pad - e,), n, jnp.int32)])
    nchunk = e_pad // NW // CHUNK

    deg_k = _make_deg_kernel(nchunk)
    edge_k = _make_edge_kernel(nchunk)

    degp = deg_k(dst)

    bp2 = bp.reshape(1, -1)
    bg1_2 = bg1.reshape(1, -1)
    bg2_2 = bg2.reshape(1, -1)
    bm1_2 = bm1.reshape(1, -1)
    bm2_2 = bm2.reshape(1, -1)
    g1_2 = g1.reshape(1, -1)
    g2_2 = g2.reshape(1, -1)
    be1_2 = be1.reshape(1, -1)
    be2_2 = be2.reshape(1, -1)
    bm3_2 = bm3.reshape(1, 1)
    bpred_2 = bpred.reshape(1, 1)

    mo, m1p, dinv = pl.pallas_call(
        _pre_body,
        grid=grid,
        in_specs=[
            _row_spec(D), _part_spec(16),
            _full_spec((D, D)), _full_spec((1, D)), _full_spec((D, D)),
            _full_spec((D, D)), _full_spec((1, D)), _full_spec((1, D)),
            _full_spec((1, D)),
            _full_spec((D, D)), _full_spec((1, D)), _full_spec((1, D)),
            _full_spec((1, D)),
            _full_spec((D, 1)), _full_spec((1, 1)),
        ],
        out_specs=[_row_spec(1), _row_spec(D), _row_spec(1)],
        out_shape=[
            jax.ShapeDtypeStruct((n, 1), jnp.float32),
            jax.ShapeDtypeStruct((n, D), jnp.float32),
            jax.ShapeDtypeStruct((n, 1), jnp.float32),
        ],
    )(x, degp, Wp, bp2, Wg1, Wm1, bm1_2, g1_2, be1_2,
      Wm2, bm2_2, g2_2, be2_2, Wm3, bm3_2)

    p1 = edge_k(m1p, src, dst)

    h1, m2p = pl.pallas_call(
        _mid_body,
        grid=grid,
        in_specs=[
            _part_spec(D), _row_spec(D), _row_spec(1),
            _full_spec((1, D)), _full_spec((D, D)),
        ],
        out_specs=[_row_spec(D), _row_spec(D)],
        out_shape=[
            jax.ShapeDtypeStruct((n, D), jnp.float32),
            jax.ShapeDtypeStruct((n, D), jnp.float32),
        ],
    )(p1, m1p, dinv, bg1_2, Wg2)

    p2 = edge_k(m2p, src, dst)

    out = pl.pallas_call(
        _post_body,
        grid=grid,
        in_specs=[
            _part_spec(D), _row_spec(D), _row_spec(1),
            _full_spec((1, D)), _row_spec(D),
            _full_spec((D, 1)), _full_spec((1, 1)), _row_spec(1),
        ],
        out_specs=[_row_spec(1)],
        out_shape=[jax.ShapeDtypeStruct((n, 1), jnp.float32)],
    )(p2, m2p, dinv, bg2_2, h1, Wpred, bpred_2, mo)[0]

    return out
